# baseline (speedup 1.0000x reference)
import functools

import jax
import jax.numpy as jnp
from jax import lax
from jax.experimental import pallas as pl
from jax.experimental.pallas import tpu as pltpu
from jax.experimental.pallas import tpu_sc as plsc

N = 10000
NA = 10008
E = 320000
D = 128

NC = 2
NS = 16
NW = NC * NS
EPW = E // NW
K = 128
EPWP = 10240
HCHUNK = 40
NCHUNK = 2 * HCHUNK
RPT = 640
CPY = 80
NCPY = RPT // CPY
NCPY_LAST = (N - 15 * RPT) // CPY


def _sc_agg_body(h_hbm, src_hbm, dst_hbm, out_hbm,
                 acc_sh, src_v, dst_v, rows0_v, rows1_v,
                 gsemA, gsemB, isem):
    c = lax.axis_index("c")
    s = lax.axis_index("s")
    wid = c * NS + s

    idma_s = pltpu.async_copy(src_hbm.at[wid, 0], src_v, isem)
    idma_d = pltpu.async_copy(dst_hbm.at[wid, 0], dst_v, isem)

    buf0 = rows0_v.at[pl.ds(0, CPY), :]
    buf1 = rows1_v.at[pl.ds(0, CPY), :]

    def init_from_h(ncpy):
        for j in range(ncpy):
            r0 = s * RPT + j * CPY
            b = buf0 if j % 2 == 0 else buf1
            pltpu.sync_copy(h_hbm.at[pl.ds(r0, CPY), :], b)
            pltpu.sync_copy(b, acc_sh.at[pl.ds(r0, CPY), :])

    def init_zero(ncpy):
        def zrow(r, carry):
            for cc in range(D // 16):
                rows0_v[r, pl.ds(cc * 16, 16)] = jnp.zeros((16,), jnp.float32)
            return carry
        lax.fori_loop(0, CPY, zrow, 0)
        for j in range(ncpy):
            r0 = s * RPT + j * CPY
            pltpu.sync_copy(buf0, acc_sh.at[pl.ds(r0, CPY), :])

    is_last = s == NS - 1

    @pl.when(jnp.logical_and(c == 0, jnp.logical_not(is_last)))
    def _():
        init_from_h(NCPY)

    @pl.when(jnp.logical_and(c == 0, is_last))
    def _():
        init_from_h(NCPY_LAST)
        def zrow(r, carry):
            for cc in range(D // 16):
                rows0_v[r, pl.ds(cc * 16, 16)] = jnp.zeros((16,), jnp.float32)
            return carry
        lax.fori_loop(0, NA - N, zrow, 0)
        pltpu.sync_copy(rows0_v.at[pl.ds(0, NA - N), :],
                        acc_sh.at[pl.ds(N, NA - N), :])

    @pl.when(jnp.logical_and(c != 0, jnp.logical_not(is_last)))
    def _():
        init_zero(NCPY)

    @pl.when(jnp.logical_and(c != 0, is_last))
    def _():
        init_zero(NCPY_LAST)
        pltpu.sync_copy(rows0_v.at[pl.ds(0, NA - N), :],
                        acc_sh.at[pl.ds(N, NA - N), :])

    plsc.subcore_barrier()

    idma_s.wait()
    idma_d.wait()

    def gstart(j, rows, sem):
        pltpu.async_copy(h_hbm.at[src_v.at[j]], rows, sem)

    def gwait(rows, sem):
        pltpu.make_async_copy(h_hbm.at[src_v.at[0]], rows, sem).wait()

    def chunk2(j2, carry):
        gwait(rows0_v, gsemA)
        gstart(j2 + 1, rows1_v, gsemB)
        pltpu.sync_copy(rows0_v, acc_sh.at[dst_v.at[j2]], add=True)
        gwait(rows1_v, gsemB)
        jn = jnp.minimum(j2 + 2, HCHUNK - 1)
        gstart(jn, rows0_v, gsemA)
        pltpu.sync_copy(rows1_v, acc_sh.at[dst_v.at[j2 + 1]], add=True)
        return carry

    def run_half():
        gstart(0, rows0_v, gsemA)
        lax.fori_loop(0, HCHUNK // 2, lambda i, cy: chunk2(i * 2, cy), 0)
        gwait(rows0_v, gsemA)

    run_half()
    pltpu.async_copy(src_hbm.at[wid, 1], src_v, isem).wait()
    pltpu.async_copy(dst_hbm.at[wid, 1], dst_v, isem).wait()
    run_half()

    plsc.subcore_barrier()

    def copy_out(ncpy):
        for j in range(ncpy):
            r0 = s * RPT + j * CPY
            b = buf0 if j % 2 == 0 else buf1
            pltpu.sync_copy(acc_sh.at[pl.ds(r0, CPY), :], b)
            pltpu.sync_copy(b, out_hbm.at[c, pl.ds(r0, CPY), :])

    @pl.when(jnp.logical_not(is_last))
    def _():
        copy_out(NCPY)

    @pl.when(is_last)
    def _():
        copy_out(NCPY_LAST)


_sc_agg = pl.kernel(
    _sc_agg_body,
    out_type=jax.ShapeDtypeStruct((NC, N, D), jnp.float32),
    mesh=plsc.VectorSubcoreMesh(core_axis_name="c", subcore_axis_name="s",
                                num_cores=NC, num_subcores=NS),
    scratch_types=[
        pltpu.VMEM_SHARED((NA, D), jnp.float32),
        pltpu.VMEM((HCHUNK, K), jnp.int32),
        pltpu.VMEM((HCHUNK, K), jnp.int32),
        pltpu.VMEM((K, D), jnp.float32),
        pltpu.VMEM((K, D), jnp.float32),
        pltpu.SemaphoreType.DMA,
        pltpu.SemaphoreType.DMA,
        pltpu.SemaphoreType.DMA,
    ],
)

BN = 1000


def _mlp_body(relu_out, p_ref, w1_ref, b1_ref, w2_ref, b2_ref, o_ref):
    z = p_ref[0] + p_ref[1]
    t = jnp.maximum(
        jnp.dot(z, w1_ref[...], preferred_element_type=jnp.float32)
        + b1_ref[...], 0.0)
    o = jnp.dot(t, w2_ref[...], preferred_element_type=jnp.float32) + b2_ref[...]
    if relu_out:
        o = jnp.maximum(o, 0.0)
    o_ref[...] = o


def _mlp(p, w1, b1, w2, b2, relu_out):
    return pl.pallas_call(
        functools.partial(_mlp_body, relu_out),
        grid=(N // BN,),
        in_specs=[
            pl.BlockSpec((NC, BN, D), lambda i: (0, i, 0)),
            pl.BlockSpec((D, D), lambda i: (0, 0)),
            pl.BlockSpec((1, D), lambda i: (0, 0)),
            pl.BlockSpec((D, D), lambda i: (0, 0)),
            pl.BlockSpec((1, D), lambda i: (0, 0)),
        ],
        out_specs=pl.BlockSpec((BN, D), lambda i: (i, 0)),
        out_shape=jax.ShapeDtypeStruct((N, D), jnp.float32),
    )(p, w1, b1.reshape(1, D), w2, b2.reshape(1, D))


def kernel(x, edge_index, W1_0, b1_0, W2_0, b2_0, W1_1, b1_1, W2_1, b2_1):
    pad = EPWP - EPW
    src = jnp.pad(edge_index[0].reshape(NW, EPW), ((0, 0), (0, pad)))
    src = src.reshape(NW, 2, HCHUNK, K)
    dst = jnp.pad(edge_index[1].reshape(NW, EPW), ((0, 0), (0, pad)),
                  constant_values=N)
    dst = dst.reshape(NW, 2, HCHUNK, K)
    p = _sc_agg(x, src, dst)
    h = _mlp(p, W1_0, b1_0, W2_0, b2_0, relu_out=True)
    q = _sc_agg(h, src, dst)
    out = _mlp(q, W1_1, b1_1, W2_1, b2_1, relu_out=False)
    return out


# speedup vs baseline: 1.2350x; 1.2350x over previous
import functools

import jax
import jax.numpy as jnp
from jax import lax
from jax.experimental import pallas as pl
from jax.experimental.pallas import tpu as pltpu
from jax.experimental.pallas import tpu_sc as plsc

N = 10000
NA = 10008
E = 320000
D = 128

NC = 2
NS = 16
NW = NC * NS
EPW = E // NW
K = 128
EPWP = 10240
HCHUNK = 40
NCHUNK = 2 * HCHUNK
RPT = 640
CPY = 80
NCPY = RPT // CPY
NCPY_LAST = (N - 15 * RPT) // CPY


def _sc_agg_body(h_hbm, src_hbm, dst_hbm, out_hbm,
                 acc_sh, src_v, dst_v, rows0_v, rows1_v,
                 gsemA, gsemB, isem):
    c = lax.axis_index("c")
    s = lax.axis_index("s")
    wid = c * NS + s

    idma_s = pltpu.async_copy(src_hbm.at[wid, 0], src_v, isem)
    idma_d = pltpu.async_copy(dst_hbm.at[wid, 0], dst_v, isem)

    buf0 = rows0_v.at[pl.ds(0, CPY), :]
    buf1 = rows1_v.at[pl.ds(0, CPY), :]

    def init_from_h(ncpy):
        for j in range(ncpy):
            r0 = s * RPT + j * CPY
            b = buf0 if j % 2 == 0 else buf1
            pltpu.sync_copy(h_hbm.at[pl.ds(r0, CPY), :], b)
            pltpu.sync_copy(b, acc_sh.at[pl.ds(r0, CPY), :])

    def init_zero(ncpy):
        def zrow(r, carry):
            for cc in range(D // 16):
                rows0_v[r, pl.ds(cc * 16, 16)] = jnp.zeros((16,), jnp.float32)
            return carry
        lax.fori_loop(0, CPY, zrow, 0)
        for j in range(ncpy):
            r0 = s * RPT + j * CPY
            pltpu.sync_copy(buf0, acc_sh.at[pl.ds(r0, CPY), :])

    is_last = s == NS - 1

    @pl.when(jnp.logical_and(c == 0, jnp.logical_not(is_last)))
    def _():
        init_from_h(NCPY)

    @pl.when(jnp.logical_and(c == 0, is_last))
    def _():
        init_from_h(NCPY_LAST)
        def zrow(r, carry):
            for cc in range(D // 16):
                rows0_v[r, pl.ds(cc * 16, 16)] = jnp.zeros((16,), jnp.float32)
            return carry
        lax.fori_loop(0, NA - N, zrow, 0)
        pltpu.sync_copy(rows0_v.at[pl.ds(0, NA - N), :],
                        acc_sh.at[pl.ds(N, NA - N), :])

    @pl.when(jnp.logical_and(c != 0, jnp.logical_not(is_last)))
    def _():
        init_zero(NCPY)

    @pl.when(jnp.logical_and(c != 0, is_last))
    def _():
        init_zero(NCPY_LAST)
        pltpu.sync_copy(rows0_v.at[pl.ds(0, NA - N), :],
                        acc_sh.at[pl.ds(N, NA - N), :])

    plsc.subcore_barrier()

    idma_s.wait()
    idma_d.wait()

    def gstart(j, rows, sem):
        pltpu.async_copy(h_hbm.at[src_v.at[j]], rows, sem)

    def gwait(rows, sem):
        pltpu.make_async_copy(h_hbm.at[src_v.at[0]], rows, sem).wait()

    def chunk1(j, carry):
        gstart(j, rows0_v, gsemA)
        gwait(rows0_v, gsemA)
        pltpu.sync_copy(rows0_v, acc_sh.at[dst_v.at[j]], add=True)
        return carry

    def run_half():
        lax.fori_loop(0, HCHUNK, chunk1, 0)

    run_half()
    pltpu.async_copy(src_hbm.at[wid, 1], src_v, isem).wait()
    pltpu.async_copy(dst_hbm.at[wid, 1], dst_v, isem).wait()
    run_half()

    plsc.subcore_barrier()

    def copy_out(ncpy):
        for j in range(ncpy):
            r0 = s * RPT + j * CPY
            b = buf0 if j % 2 == 0 else buf1
            pltpu.sync_copy(acc_sh.at[pl.ds(r0, CPY), :], b)
            pltpu.sync_copy(b, out_hbm.at[c, pl.ds(r0, CPY), :])

    @pl.when(jnp.logical_not(is_last))
    def _():
        copy_out(NCPY)

    @pl.when(is_last)
    def _():
        copy_out(NCPY_LAST)


_sc_agg = pl.kernel(
    _sc_agg_body,
    out_type=jax.ShapeDtypeStruct((NC, N, D), jnp.float32),
    mesh=plsc.VectorSubcoreMesh(core_axis_name="c", subcore_axis_name="s",
                                num_cores=NC, num_subcores=NS),
    scratch_types=[
        pltpu.VMEM_SHARED((NA, D), jnp.float32),
        pltpu.VMEM((HCHUNK, K), jnp.int32),
        pltpu.VMEM((HCHUNK, K), jnp.int32),
        pltpu.VMEM((K, D), jnp.float32),
        pltpu.VMEM((K, D), jnp.float32),
        pltpu.SemaphoreType.DMA,
        pltpu.SemaphoreType.DMA,
        pltpu.SemaphoreType.DMA,
    ],
)

BN = 1000


def _mlp_body(relu_out, p_ref, w1_ref, b1_ref, w2_ref, b2_ref, o_ref):
    z = p_ref[0] + p_ref[1]
    t = jnp.maximum(
        jnp.dot(z, w1_ref[...], preferred_element_type=jnp.float32)
        + b1_ref[...], 0.0)
    o = jnp.dot(t, w2_ref[...], preferred_element_type=jnp.float32) + b2_ref[...]
    if relu_out:
        o = jnp.maximum(o, 0.0)
    o_ref[...] = o


def _mlp(p, w1, b1, w2, b2, relu_out):
    return pl.pallas_call(
        functools.partial(_mlp_body, relu_out),
        grid=(N // BN,),
        in_specs=[
            pl.BlockSpec((NC, BN, D), lambda i: (0, i, 0)),
            pl.BlockSpec((D, D), lambda i: (0, 0)),
            pl.BlockSpec((1, D), lambda i: (0, 0)),
            pl.BlockSpec((D, D), lambda i: (0, 0)),
            pl.BlockSpec((1, D), lambda i: (0, 0)),
        ],
        out_specs=pl.BlockSpec((BN, D), lambda i: (i, 0)),
        out_shape=jax.ShapeDtypeStruct((N, D), jnp.float32),
    )(p, w1, b1.reshape(1, D), w2, b2.reshape(1, D))


def kernel(x, edge_index, W1_0, b1_0, W2_0, b2_0, W1_1, b1_1, W2_1, b2_1):
    pad = EPWP - EPW
    src = jnp.pad(edge_index[0].reshape(NW, EPW), ((0, 0), (0, pad)))
    src = src.reshape(NW, 2, HCHUNK, K)
    dst = jnp.pad(edge_index[1].reshape(NW, EPW), ((0, 0), (0, pad)),
                  constant_values=N)
    dst = dst.reshape(NW, 2, HCHUNK, K)
    p = _sc_agg(x, src, dst)
    h = _mlp(p, W1_0, b1_0, W2_0, b2_0, relu_out=True)
    q = _sc_agg(h, src, dst)
    out = _mlp(q, W1_1, b1_1, W2_1, b2_1, relu_out=False)
    return out


# speedup vs baseline: 3.1983x; 2.5898x over previous
import functools

import jax
import jax.numpy as jnp
from jax import lax
from jax.experimental import pallas as pl
from jax.experimental.pallas import tpu as pltpu
from jax.experimental.pallas import tpu_sc as plsc

N = 10000
E = 320000
D = 128

NC = 2
NS = 16
NW = NC * NS
EPW = E // NW
K = 80
NCHUNK = EPW // K
NPAIR = (NCHUNK - 1) // 2
RPT = 640
CPY = 80
NCPY = RPT // CPY
NCPY_LAST = (N - 15 * RPT) // CPY


def _sc_agg_body(h_hbm, src_hbm, dst_hbm, out_hbm,
                 acc_sh, srcA, dstA, srcB, dstB, rowsA, rowsB,
                 isemA, isemB, gsemA, gsemB):
    c = lax.axis_index("c")
    s = lax.axis_index("s")
    wid = c * NS + s
    base = wid * EPW

    buf0 = rowsA.at[pl.ds(0, CPY), :]
    buf1 = rowsB.at[pl.ds(0, CPY), :]

    def init_from_h(ncpy):
        for j in range(ncpy):
            r0 = s * RPT + j * CPY
            b = buf0 if j % 2 == 0 else buf1
            pltpu.sync_copy(h_hbm.at[pl.ds(r0, CPY), :], b)
            pltpu.sync_copy(b, acc_sh.at[pl.ds(r0, CPY), :])

    def init_zero(ncpy):
        def zrow(r, carry):
            for cc in range(D // 16):
                rowsA[r, pl.ds(cc * 16, 16)] = jnp.zeros((16,), jnp.float32)
            return carry
        lax.fori_loop(0, CPY, zrow, 0)
        for j in range(ncpy):
            r0 = s * RPT + j * CPY
            pltpu.sync_copy(buf0, acc_sh.at[pl.ds(r0, CPY), :])

    is_last = s == NS - 1

    @pl.when(jnp.logical_and(c == 0, jnp.logical_not(is_last)))
    def _():
        init_from_h(NCPY)

    @pl.when(jnp.logical_and(c == 0, is_last))
    def _():
        init_from_h(NCPY_LAST)

    @pl.when(jnp.logical_and(c != 0, jnp.logical_not(is_last)))
    def _():
        init_zero(NCPY)

    @pl.when(jnp.logical_and(c != 0, is_last))
    def _():
        init_zero(NCPY_LAST)

    plsc.subcore_barrier()

    def istart(sv, dv, isem, j):
        off = base + j * K
        pltpu.async_copy(src_hbm.at[pl.ds(off, K)], sv, isem)
        pltpu.async_copy(dst_hbm.at[pl.ds(off, K)], dv, isem)

    def iwait(sv, dv, isem):
        pltpu.make_async_copy(src_hbm.at[pl.ds(0, K)], sv, isem).wait()
        pltpu.make_async_copy(dst_hbm.at[pl.ds(0, K)], dv, isem).wait()

    def gstart(sv, rows, gsem):
        pltpu.async_copy(h_hbm.at[sv], rows, gsem)

    def gwait(rows, gsem):
        pltpu.make_async_copy(h_hbm.at[pl.ds(0, K), :], rows, gsem).wait()

    def scat(rows, dv):
        pltpu.sync_copy(rows, acc_sh.at[dv], add=True)

    istart(srcA, dstA, isemA, 0)
    iwait(srcA, dstA, isemA)
    gstart(srcA, rowsA, gsemA)
    istart(srcB, dstB, isemB, 1)

    def pair(i, carry):
        j2 = i * 2
        gwait(rowsA, gsemA)
        iwait(srcB, dstB, isemB)
        gstart(srcB, rowsB, gsemB)
        scat(rowsA, dstA)
        istart(srcA, dstA, isemA, jnp.minimum(j2 + 2, NCHUNK - 1))
        gwait(rowsB, gsemB)
        iwait(srcA, dstA, isemA)
        gstart(srcA, rowsA, gsemA)
        scat(rowsB, dstB)
        istart(srcB, dstB, isemB, jnp.minimum(j2 + 3, NCHUNK - 1))
        return carry

    lax.fori_loop(0, NPAIR, pair, 0)

    gwait(rowsA, gsemA)
    scat(rowsA, dstA)
    iwait(srcB, dstB, isemB)

    plsc.subcore_barrier()

    def copy_out(ncpy):
        for j in range(ncpy):
            r0 = s * RPT + j * CPY
            b = buf0 if j % 2 == 0 else buf1
            pltpu.sync_copy(acc_sh.at[pl.ds(r0, CPY), :], b)
            pltpu.sync_copy(b, out_hbm.at[c, pl.ds(r0, CPY), :])

    @pl.when(jnp.logical_not(is_last))
    def _():
        copy_out(NCPY)

    @pl.when(is_last)
    def _():
        copy_out(NCPY_LAST)


_sc_agg = pl.kernel(
    _sc_agg_body,
    out_type=jax.ShapeDtypeStruct((NC, N, D), jnp.float32),
    mesh=plsc.VectorSubcoreMesh(core_axis_name="c", subcore_axis_name="s",
                                num_cores=NC, num_subcores=NS),
    scratch_types=[
        pltpu.VMEM_SHARED((N, D), jnp.float32),
        pltpu.VMEM((K,), jnp.int32),
        pltpu.VMEM((K,), jnp.int32),
        pltpu.VMEM((K,), jnp.int32),
        pltpu.VMEM((K,), jnp.int32),
        pltpu.VMEM((K, D), jnp.float32),
        pltpu.VMEM((K, D), jnp.float32),
        pltpu.SemaphoreType.DMA,
        pltpu.SemaphoreType.DMA,
        pltpu.SemaphoreType.DMA,
        pltpu.SemaphoreType.DMA,
    ],
)

BN = 1000


def _mlp_body(relu_out, p_ref, w1_ref, b1_ref, w2_ref, b2_ref, o_ref):
    z = p_ref[0] + p_ref[1]
    t = jnp.maximum(
        jnp.dot(z, w1_ref[...], preferred_element_type=jnp.float32)
        + b1_ref[...], 0.0)
    o = jnp.dot(t, w2_ref[...], preferred_element_type=jnp.float32) + b2_ref[...]
    if relu_out:
        o = jnp.maximum(o, 0.0)
    o_ref[...] = o


def _mlp(p, w1, b1, w2, b2, relu_out):
    return pl.pallas_call(
        functools.partial(_mlp_body, relu_out),
        grid=(N // BN,),
        in_specs=[
            pl.BlockSpec((NC, BN, D), lambda i: (0, i, 0)),
            pl.BlockSpec((D, D), lambda i: (0, 0)),
            pl.BlockSpec((1, D), lambda i: (0, 0)),
            pl.BlockSpec((D, D), lambda i: (0, 0)),
            pl.BlockSpec((1, D), lambda i: (0, 0)),
        ],
        out_specs=pl.BlockSpec((BN, D), lambda i: (i, 0)),
        out_shape=jax.ShapeDtypeStruct((N, D), jnp.float32),
    )(p, w1, b1.reshape(1, D), w2, b2.reshape(1, D))


def kernel(x, edge_index, W1_0, b1_0, W2_0, b2_0, W1_1, b1_1, W2_1, b2_1):
    src = edge_index[0]
    dst = edge_index[1]
    p = _sc_agg(x, src, dst)
    h = _mlp(p, W1_0, b1_0, W2_0, b2_0, relu_out=True)
    q = _sc_agg(h, src, dst)
    out = _mlp(q, W1_1, b1_1, W2_1, b2_1, relu_out=False)
    return out


# speedup vs baseline: 3.2938x; 1.0299x over previous
import functools

import jax
import jax.numpy as jnp
from jax import lax
from jax.experimental import pallas as pl
from jax.experimental.pallas import tpu as pltpu
from jax.experimental.pallas import tpu_sc as plsc

N = 10000
E = 320000
D = 128

NC = 2
NS = 16
NW = NC * NS
EPW = E // NW
K = 80
NCHUNK = EPW // K
NPAIR = (NCHUNK - 1) // 2
RPT = 640
CPY = 80
NCPY = RPT // CPY
NCPY_LAST = (N - 15 * RPT) // CPY


def _sc_agg_body(h_hbm, src_hbm, dst_hbm, out_hbm,
                 acc_sh, srcA, dstA, srcB, dstB, rowsA, rowsB,
                 isemA, isemB, gsemA, gsemB):
    c = lax.axis_index("c")
    s = lax.axis_index("s")
    wid = c * NS + s
    base = wid * EPW

    istart0 = pltpu.async_copy(src_hbm.at[pl.ds(base, K)], srcA, isemA)
    istart0b = pltpu.async_copy(dst_hbm.at[pl.ds(base, K)], dstA, isemA)
    istart1 = pltpu.async_copy(src_hbm.at[pl.ds(base + K, K)], srcB, isemB)
    istart1b = pltpu.async_copy(dst_hbm.at[pl.ds(base + K, K)], dstB, isemB)

    buf0 = rowsA.at[pl.ds(0, CPY), :]
    buf1 = rowsB.at[pl.ds(0, CPY), :]

    def init_from_h(ncpy):
        def hload(j, b, sem):
            pltpu.async_copy(h_hbm.at[pl.ds(s * RPT + j * CPY, CPY), :], b, sem)

        def hwait(b, sem):
            pltpu.make_async_copy(h_hbm.at[pl.ds(0, CPY), :], b, sem).wait()

        hload(0, buf0, gsemA)
        for j in range(ncpy):
            b, sem = (buf0, gsemA) if j % 2 == 0 else (buf1, gsemB)
            nb, nsem = (buf1, gsemB) if j % 2 == 0 else (buf0, gsemA)
            hwait(b, sem)
            if j + 1 < ncpy:
                hload(j + 1, nb, nsem)
            pltpu.sync_copy(b, acc_sh.at[pl.ds(s * RPT + j * CPY, CPY), :])

    def init_zero(ncpy):
        def zrow(r, carry):
            for cc in range(D // 16):
                rowsA[r, pl.ds(cc * 16, 16)] = jnp.zeros((16,), jnp.float32)
            return carry
        lax.fori_loop(0, CPY, zrow, 0)
        for j in range(ncpy):
            r0 = s * RPT + j * CPY
            pltpu.async_copy(buf0, acc_sh.at[pl.ds(r0, CPY), :], gsemA)
        for j in range(ncpy):
            pltpu.make_async_copy(buf0, acc_sh.at[pl.ds(0, CPY), :], gsemA).wait()

    is_last = s == NS - 1

    @pl.when(jnp.logical_and(c == 0, jnp.logical_not(is_last)))
    def _():
        init_from_h(NCPY)

    @pl.when(jnp.logical_and(c == 0, is_last))
    def _():
        init_from_h(NCPY_LAST)

    @pl.when(jnp.logical_and(c != 0, jnp.logical_not(is_last)))
    def _():
        init_zero(NCPY)

    @pl.when(jnp.logical_and(c != 0, is_last))
    def _():
        init_zero(NCPY_LAST)

    plsc.subcore_barrier()

    def istart(sv, dv, isem, j):
        off = base + j * K
        pltpu.async_copy(src_hbm.at[pl.ds(off, K)], sv, isem)
        pltpu.async_copy(dst_hbm.at[pl.ds(off, K)], dv, isem)

    def iwait(sv, dv, isem):
        pltpu.make_async_copy(src_hbm.at[pl.ds(0, K)], sv, isem).wait()
        pltpu.make_async_copy(dst_hbm.at[pl.ds(0, K)], dv, isem).wait()

    def gstart(sv, rows, gsem):
        pltpu.async_copy(h_hbm.at[sv], rows, gsem)

    def gwait(rows, gsem):
        pltpu.make_async_copy(h_hbm.at[pl.ds(0, K), :], rows, gsem).wait()

    def scat(rows, dv):
        pltpu.sync_copy(rows, acc_sh.at[dv], add=True)

    iwait(srcA, dstA, isemA)
    gstart(srcA, rowsA, gsemA)

    def pair(i, carry):
        j2 = i * 2
        gwait(rowsA, gsemA)
        iwait(srcB, dstB, isemB)
        gstart(srcB, rowsB, gsemB)
        scat(rowsA, dstA)
        istart(srcA, dstA, isemA, jnp.minimum(j2 + 2, NCHUNK - 1))
        gwait(rowsB, gsemB)
        iwait(srcA, dstA, isemA)
        gstart(srcA, rowsA, gsemA)
        scat(rowsB, dstB)
        istart(srcB, dstB, isemB, jnp.minimum(j2 + 3, NCHUNK - 1))
        return carry

    lax.fori_loop(0, NPAIR, pair, 0)

    gwait(rowsA, gsemA)
    scat(rowsA, dstA)
    iwait(srcB, dstB, isemB)

    plsc.subcore_barrier()

    def copy_out(ncpy):
        def aread(j, b, sem):
            pltpu.async_copy(acc_sh.at[pl.ds(s * RPT + j * CPY, CPY), :], b, sem)

        def await_(b, sem):
            pltpu.make_async_copy(acc_sh.at[pl.ds(0, CPY), :], b, sem).wait()

        aread(0, buf0, gsemA)
        for j in range(ncpy):
            b, sem = (buf0, gsemA) if j % 2 == 0 else (buf1, gsemB)
            nb, nsem = (buf1, gsemB) if j % 2 == 0 else (buf0, gsemA)
            await_(b, sem)
            if j + 1 < ncpy:
                aread(j + 1, nb, nsem)
            pltpu.sync_copy(b, out_hbm.at[c, pl.ds(s * RPT + j * CPY, CPY), :])

    @pl.when(jnp.logical_not(is_last))
    def _():
        copy_out(NCPY)

    @pl.when(is_last)
    def _():
        copy_out(NCPY_LAST)


_sc_agg = pl.kernel(
    _sc_agg_body,
    out_type=jax.ShapeDtypeStruct((NC, N, D), jnp.float32),
    mesh=plsc.VectorSubcoreMesh(core_axis_name="c", subcore_axis_name="s",
                                num_cores=NC, num_subcores=NS),
    scratch_types=[
        pltpu.VMEM_SHARED((N, D), jnp.float32),
        pltpu.VMEM((K,), jnp.int32),
        pltpu.VMEM((K,), jnp.int32),
        pltpu.VMEM((K,), jnp.int32),
        pltpu.VMEM((K,), jnp.int32),
        pltpu.VMEM((K, D), jnp.float32),
        pltpu.VMEM((K, D), jnp.float32),
        pltpu.SemaphoreType.DMA,
        pltpu.SemaphoreType.DMA,
        pltpu.SemaphoreType.DMA,
        pltpu.SemaphoreType.DMA,
    ],
)

BN = 1000


def _mlp_body(relu_out, p_ref, w1_ref, b1_ref, w2_ref, b2_ref, o_ref):
    z = p_ref[0] + p_ref[1]
    t = jnp.maximum(
        jnp.dot(z, w1_ref[...], preferred_element_type=jnp.float32)
        + b1_ref[...], 0.0)
    o = jnp.dot(t, w2_ref[...], preferred_element_type=jnp.float32) + b2_ref[...]
    if relu_out:
        o = jnp.maximum(o, 0.0)
    o_ref[...] = o


def _mlp(p, w1, b1, w2, b2, relu_out):
    return pl.pallas_call(
        functools.partial(_mlp_body, relu_out),
        grid=(N // BN,),
        in_specs=[
            pl.BlockSpec((NC, BN, D), lambda i: (0, i, 0)),
            pl.BlockSpec((D, D), lambda i: (0, 0)),
            pl.BlockSpec((1, D), lambda i: (0, 0)),
            pl.BlockSpec((D, D), lambda i: (0, 0)),
            pl.BlockSpec((1, D), lambda i: (0, 0)),
        ],
        out_specs=pl.BlockSpec((BN, D), lambda i: (i, 0)),
        out_shape=jax.ShapeDtypeStruct((N, D), jnp.float32),
    )(p, w1, b1.reshape(1, D), w2, b2.reshape(1, D))


def kernel(x, edge_index, W1_0, b1_0, W2_0, b2_0, W1_1, b1_1, W2_1, b2_1):
    src = edge_index[0]
    dst = edge_index[1]
    p = _sc_agg(x, src, dst)
    h = _mlp(p, W1_0, b1_0, W2_0, b2_0, relu_out=True)
    q = _sc_agg(h, src, dst)
    out = _mlp(q, W1_1, b1_1, W2_1, b2_1, relu_out=False)
    return out
